# baseline (speedup 1.0000x reference)
import dataclasses
import functools

import jax
import jax.numpy as jnp
from jax import lax
from jax.experimental import pallas as pl
from jax.experimental.pallas import tpu as pltpu
from jax.experimental.pallas import tpu_sc as plsc

V = 1000
NE = 128
NTOK = 32 * 2048
NWORK = 32
TPW = NTOK // NWORK
WIN = 32
NWIN = TPW // WIN
L = 16

ROWBLK = 40


VP = 1024


def _table_body(emb_ref, w_ref, b_ref, fused_ref):
    logits = jnp.dot(emb_ref[...], w_ref[...],
                     preferred_element_type=jnp.float32) + b_ref[...]
    m = jnp.max(logits, axis=1, keepdims=True)
    lse = m + jnp.log(jnp.sum(jnp.exp(logits - m), axis=1, keepdims=True))
    pad = jnp.zeros((ROWBLK, VP - V - 1), jnp.float32)
    fused_ref[...] = jnp.concatenate([logits, lse, pad], axis=1)


def _make_table(emb_table, W, b):
    return pl.pallas_call(
        _table_body,
        grid=(V // ROWBLK,),
        in_specs=[
            pl.BlockSpec((ROWBLK, NE), lambda i: (i, 0)),
            pl.BlockSpec((NE, V), lambda i: (0, 0)),
            pl.BlockSpec((1, V), lambda i: (0, 0)),
        ],
        out_specs=[
            pl.BlockSpec((ROWBLK, VP), lambda i: (i, 0)),
        ],
        out_shape=[
            jax.ShapeDtypeStruct((V, VP), jnp.float32),
        ],
    )(emb_table, W, b.reshape(1, V))[0]


def _sc_gather(fused, idx, tgt):
    mesh = plsc.VectorSubcoreMesh(core_axis_name="c", subcore_axis_name="s")
    cp = pltpu.CompilerParams(needs_layout_passes=False,
                              use_tc_tiling_on_sc=True)

    @functools.partial(
        pl.kernel,
        compiler_params=cp,
        out_type=(
            jax.ShapeDtypeStruct((NTOK, V), jnp.float32),
            jax.ShapeDtypeStruct((NWORK, L), jnp.float32),
        ),
        mesh=mesh,
        scratch_types=[
            pltpu.VMEM((TPW,), jnp.int32),
            pltpu.VMEM((TPW,), jnp.int32),
            pltpu.VMEM((WIN, VP), jnp.float32),
            pltpu.VMEM((WIN, V - 128 * (V // 128)), jnp.float32),
            pltpu.VMEM((L,), jnp.float32),
            pltpu.SemaphoreType.DMA,
        ],
    )
    def k(fused_hbm, idx_hbm, tgt_hbm, out_hbm, part_hbm,
          idx_v, tgt_v, rows, esrc, acc_v, gsem):
        wid = lax.axis_index("s") * 2 + lax.axis_index("c")
        base = wid * TPW
        pltpu.sync_copy(idx_hbm.at[pl.ds(base, TPW)], idx_v)
        pltpu.sync_copy(tgt_hbm.at[pl.ds(base, TPW)], tgt_v)
        acc_v[...] = jnp.zeros((L,), jnp.float32)

        @pl.loop(0, NWIN)
        def _(w):
            woff = w * WIN
            pltpu.async_copy(fused_hbm.at[idx_v.at[pl.ds(woff, WIN)]],
                             rows, gsem).wait()
            lse_col = jnp.full((L,), V, jnp.int32)
            for g in range(WIN // L):
                toff = woff + g * L
                tg = tgt_v[pl.ds(toff, L)]
                rowi = lax.iota(jnp.int32, L) + (g * L)
                lsev = plsc.load_gather(rows, [rowi, lse_col])
                logit_t = plsc.load_gather(rows, [rowi, tg])
                acc_v[...] = acc_v[...] + (lsev - logit_t)
            orow = pl.ds(base + woff, WIN)
            for k2 in range(V // 128):
                cs = pl.ds(128 * k2, 128)
                pltpu.sync_copy(rows.at[:, cs], out_hbm.at[orow, cs])
            ebase = 128 * (V // 128)
            ew = V - ebase

            @pl.loop(0, WIN)
            def _(r):
                for j in range(ew // L):
                    compact_cs = pl.ds(L * j, L)
                    compact_src = pl.ds(ebase + L * j, L)
                    esrc[r, compact_cs] = rows[r, compact_src]
                nfull = (ew // L) * L
                rem = ew - nfull
                if rem:
                    lanes = lax.iota(jnp.int32, L)
                    v = rows[r, pl.ds(ebase + nfull, L)]
                    cols = jnp.minimum(nfull + lanes, ew - 1)
                    rowv = jnp.broadcast_to(r, (L,)).astype(jnp.int32)
                    plsc.store_scatter(esrc, [rowv, cols], v,
                                       mask=lanes < rem)

            pltpu.sync_copy(esrc, out_hbm.at[orow, pl.ds(ebase, ew)])

        pltpu.sync_copy(acc_v, part_hbm.at[wid])

    return k(fused, idx, tgt)


def _loss_body(part_ref, out_ref):
    out_ref[...] = jnp.sum(part_ref[...], keepdims=True).reshape(1, 1) * (
        1.0 / NTOK)


def _reduce_loss(parts):
    return pl.pallas_call(
        _loss_body,
        out_shape=jax.ShapeDtypeStruct((1, 1), jnp.float32),
    )(parts)


def kernel(inputs, targets, emb_table, W, b):
    idx = inputs.reshape(-1).astype(jnp.int32)
    tgt = targets.reshape(-1).astype(jnp.int32)
    fused = _make_table(emb_table, W, b)
    logits, parts = _sc_gather(fused, idx, tgt)
    loss = _reduce_loss(parts)[0, 0]
    return (logits, loss)


# speedup vs baseline: 1.2169x; 1.2169x over previous
import dataclasses
import functools

import jax
import jax.numpy as jnp
from jax import lax
from jax.experimental import pallas as pl
from jax.experimental.pallas import tpu as pltpu
from jax.experimental.pallas import tpu_sc as plsc

V = 1000
NE = 128
NTOK = 32 * 2048
NWORK = 32
TPW = NTOK // NWORK
WIN = 32
NWIN = TPW // WIN
L = 16

ROWBLK = 40


VP = 1024


def _table_body(emb_ref, w_ref, b_ref, fused_ref):
    logits = jnp.dot(emb_ref[...], w_ref[...],
                     preferred_element_type=jnp.float32) + b_ref[...]
    m = jnp.max(logits, axis=1, keepdims=True)
    lse = m + jnp.log(jnp.sum(jnp.exp(logits - m), axis=1, keepdims=True))
    pad = jnp.zeros((ROWBLK, VP - V - 1), jnp.float32)
    fused_ref[...] = jnp.concatenate([logits, lse, pad], axis=1)


def _make_table(emb_table, W, b):
    return pl.pallas_call(
        _table_body,
        grid=(V // ROWBLK,),
        in_specs=[
            pl.BlockSpec((ROWBLK, NE), lambda i: (i, 0)),
            pl.BlockSpec((NE, V), lambda i: (0, 0)),
            pl.BlockSpec((1, V), lambda i: (0, 0)),
        ],
        out_specs=[
            pl.BlockSpec((ROWBLK, VP), lambda i: (i, 0)),
        ],
        out_shape=[
            jax.ShapeDtypeStruct((V, VP), jnp.float32),
        ],
    )(emb_table, W, b.reshape(1, V))[0]


def _sc_gather(fused, idx, tgt):
    mesh = plsc.VectorSubcoreMesh(core_axis_name="c", subcore_axis_name="s")
    cp = pltpu.CompilerParams(needs_layout_passes=False,
                              use_tc_tiling_on_sc=True)

    @functools.partial(
        pl.kernel,
        compiler_params=cp,
        out_type=(
            jax.ShapeDtypeStruct((NTOK, V), jnp.float32),
            jax.ShapeDtypeStruct((NWORK, L), jnp.float32),
        ),
        mesh=mesh,
        scratch_types=[
            pltpu.VMEM((TPW,), jnp.int32),
            pltpu.VMEM((TPW,), jnp.int32),
            pltpu.VMEM((WIN, VP), jnp.float32),
            pltpu.VMEM((WIN, VP), jnp.float32),
            pltpu.VMEM((WIN, V - 128 * (V // 128)), jnp.float32),
            pltpu.VMEM((WIN, V - 128 * (V // 128)), jnp.float32),
            pltpu.VMEM((L,), jnp.float32),
            pltpu.SemaphoreType.DMA,
            pltpu.SemaphoreType.DMA,
            pltpu.SemaphoreType.DMA,
            pltpu.SemaphoreType.DMA,
        ],
    )
    def k(fused_hbm, idx_hbm, tgt_hbm, out_hbm, part_hbm,
          idx_v, tgt_v, rows0, rows1, esrc0, esrc1, acc_v,
          g0, g1, o0, o1):
        wid = lax.axis_index("s") * 2 + lax.axis_index("c")
        base = wid * TPW
        pltpu.sync_copy(idx_hbm.at[pl.ds(base, TPW)], idx_v)
        pltpu.sync_copy(tgt_hbm.at[pl.ds(base, TPW)], tgt_v)
        acc_v[...] = jnp.zeros((L,), jnp.float32)

        EB = 128 * (V // 128)
        EW = V - EB

        def gcopy(w, buf, sem):
            return pltpu.make_async_copy(
                fused_hbm.at[idx_v.at[pl.ds(w * WIN, WIN)]], buf, sem)

        def ocopies(w, buf, ebuf, sem):
            orow = pl.ds(base + w * WIN, WIN)
            cps = []
            for k2 in range(V // 128):
                cs = pl.ds(128 * k2, 128)
                cps.append(pltpu.make_async_copy(
                    buf.at[:, cs], out_hbm.at[orow, cs], sem))
            cps.append(pltpu.make_async_copy(
                ebuf, out_hbm.at[orow, pl.ds(EB, EW)], sem))
            return cps

        def process(w, buf, ebuf):
            lse_col = jnp.full((L,), V, jnp.int32)
            for g in range(WIN // L):
                tg = tgt_v[pl.ds(w * WIN + g * L, L)]
                rowi = lax.iota(jnp.int32, L) + (g * L)
                lsev = plsc.load_gather(buf, [rowi, lse_col])
                logit_t = plsc.load_gather(buf, [rowi, tg])
                acc_v[...] = acc_v[...] + (lsev - logit_t)

            @pl.loop(0, WIN)
            def _(r):
                for j in range(EW // L):
                    ebuf[r, pl.ds(L * j, L)] = buf[r, pl.ds(EB + L * j, L)]
                nfull = (EW // L) * L
                rem = EW - nfull
                if rem:
                    lanes = lax.iota(jnp.int32, L)
                    v = buf[r, pl.ds(EB + nfull, L)]
                    cols = jnp.minimum(nfull + lanes, EW - 1)
                    rowv = jnp.broadcast_to(r, (L,)).astype(jnp.int32)
                    plsc.store_scatter(ebuf, [rowv, cols], v,
                                       mask=lanes < rem)

        gcopy(0, rows0, g0).start()
        gcopy(1, rows1, g1).start()

        @pl.loop(0, NWIN, step=2)
        def _(w):
            gcopy(w, rows0, g0).wait()
            process(w, rows0, esrc0)
            for c in ocopies(w, rows0, esrc0, o0):
                c.start()

            gcopy(w + 1, rows1, g1).wait()
            process(w + 1, rows1, esrc1)
            for c in ocopies(w + 1, rows1, esrc1, o1):
                c.start()

            @pl.when(w + 2 < NWIN)
            def _():
                for c in ocopies(w, rows0, esrc0, o0):
                    c.wait()
                gcopy(w + 2, rows0, g0).start()
                for c in ocopies(w + 1, rows1, esrc1, o1):
                    c.wait()
                gcopy(w + 3, rows1, g1).start()

        for c in ocopies(NWIN - 2, rows0, esrc0, o0):
            c.wait()
        for c in ocopies(NWIN - 1, rows1, esrc1, o1):
            c.wait()
        pltpu.sync_copy(acc_v, part_hbm.at[wid])

    return k(fused, idx, tgt)


def _loss_body(part_ref, out_ref):
    out_ref[...] = jnp.sum(part_ref[...], keepdims=True).reshape(1, 1) * (
        1.0 / NTOK)


def _reduce_loss(parts):
    return pl.pallas_call(
        _loss_body,
        out_shape=jax.ShapeDtypeStruct((1, 1), jnp.float32),
    )(parts)


def kernel(inputs, targets, emb_table, W, b):
    idx = inputs.reshape(-1).astype(jnp.int32)
    tgt = targets.reshape(-1).astype(jnp.int32)
    fused = _make_table(emb_table, W, b)
    logits, parts = _sc_gather(fused, idx, tgt)
    loss = _reduce_loss(parts)[0, 0]
    return (logits, loss)


# speedup vs baseline: 1.3149x; 1.0805x over previous
import dataclasses
import functools

import jax
import jax.numpy as jnp
from jax import lax
from jax.experimental import pallas as pl
from jax.experimental.pallas import tpu as pltpu
from jax.experimental.pallas import tpu_sc as plsc

V = 1000
NE = 128
NTOK = 32 * 2048
NWORK = 32
TPW = NTOK // NWORK
WIN = 32
NWIN = TPW // WIN
L = 16

ROWBLK = 40


VP = 1024


def _table_body(emb_ref, w_ref, b_ref, fused_ref):
    logits = jnp.dot(emb_ref[...], w_ref[...],
                     preferred_element_type=jnp.float32) + b_ref[...]
    m = jnp.max(logits, axis=1, keepdims=True)
    lse = m + jnp.log(jnp.sum(jnp.exp(logits - m), axis=1, keepdims=True))
    pad = jnp.zeros((ROWBLK, VP - V - 1), jnp.float32)
    fused_ref[...] = jnp.concatenate([logits, lse, pad], axis=1)


def _make_table(emb_table, W, b):
    return pl.pallas_call(
        _table_body,
        grid=(V // ROWBLK,),
        in_specs=[
            pl.BlockSpec((ROWBLK, NE), lambda i: (i, 0)),
            pl.BlockSpec((NE, V), lambda i: (0, 0)),
            pl.BlockSpec((1, V), lambda i: (0, 0)),
        ],
        out_specs=[
            pl.BlockSpec((ROWBLK, VP), lambda i: (i, 0)),
        ],
        out_shape=[
            jax.ShapeDtypeStruct((V, VP), jnp.float32),
        ],
    )(emb_table, W, b.reshape(1, V))[0]


def _sc_gather(fused, idx, tgt):
    mesh = plsc.VectorSubcoreMesh(core_axis_name="c", subcore_axis_name="s")
    cp = pltpu.CompilerParams(needs_layout_passes=False,
                              use_tc_tiling_on_sc=True)

    @functools.partial(
        pl.kernel,
        compiler_params=cp,
        out_type=(
            jax.ShapeDtypeStruct((NTOK, VP), jnp.float32),
            jax.ShapeDtypeStruct((NWORK, L), jnp.float32),
        ),
        mesh=mesh,
        scratch_types=[
            pltpu.VMEM((TPW,), jnp.int32),
            pltpu.VMEM((TPW,), jnp.int32),
            pltpu.VMEM((WIN, VP), jnp.float32),
            pltpu.VMEM((WIN, VP), jnp.float32),
            pltpu.VMEM((L,), jnp.float32),
            pltpu.SemaphoreType.DMA,
            pltpu.SemaphoreType.DMA,
            pltpu.SemaphoreType.DMA,
            pltpu.SemaphoreType.DMA,
        ],
    )
    def k(fused_hbm, idx_hbm, tgt_hbm, out_hbm, part_hbm,
          idx_v, tgt_v, rows0, rows1, acc_v,
          g0, g1, o0, o1):
        wid = lax.axis_index("s") * 2 + lax.axis_index("c")
        base = wid * TPW
        pltpu.sync_copy(idx_hbm.at[pl.ds(base, TPW)], idx_v)
        pltpu.sync_copy(tgt_hbm.at[pl.ds(base, TPW)], tgt_v)
        acc_v[...] = jnp.zeros((L,), jnp.float32)

        def gcopy(w, buf, sem):
            return pltpu.make_async_copy(
                fused_hbm.at[idx_v.at[pl.ds(w * WIN, WIN)]], buf, sem)

        def ocopy(w, buf, sem):
            return pltpu.make_async_copy(
                buf, out_hbm.at[pl.ds(base + w * WIN, WIN)], sem)

        def process(w, buf):
            lse_col = jnp.full((L,), V, jnp.int32)
            for g in range(WIN // L):
                tg = tgt_v[pl.ds(w * WIN + g * L, L)]
                rowi = lax.iota(jnp.int32, L) + (g * L)
                lsev = plsc.load_gather(buf, [rowi, lse_col])
                logit_t = plsc.load_gather(buf, [rowi, tg])
                acc_v[...] = acc_v[...] + (lsev - logit_t)

        gcopy(0, rows0, g0).start()
        gcopy(1, rows1, g1).start()

        @pl.loop(0, NWIN, step=2)
        def _(w):
            gcopy(w, rows0, g0).wait()
            process(w, rows0)
            ocopy(w, rows0, o0).start()

            gcopy(w + 1, rows1, g1).wait()
            process(w + 1, rows1)
            ocopy(w + 1, rows1, o1).start()

            @pl.when(w + 2 < NWIN)
            def _():
                ocopy(w, rows0, o0).wait()
                gcopy(w + 2, rows0, g0).start()
                ocopy(w + 1, rows1, o1).wait()
                gcopy(w + 3, rows1, g1).start()

        ocopy(NWIN - 2, rows0, o0).wait()
        ocopy(NWIN - 1, rows1, o1).wait()
        pltpu.sync_copy(acc_v, part_hbm.at[wid])

    return k(fused, idx, tgt)


def _loss_body(part_ref, out_ref):
    out_ref[...] = jnp.sum(part_ref[...], keepdims=True).reshape(1, 1) * (
        1.0 / NTOK)


def _reduce_loss(parts):
    return pl.pallas_call(
        _loss_body,
        out_shape=jax.ShapeDtypeStruct((1, 1), jnp.float32),
    )(parts)


def kernel(inputs, targets, emb_table, W, b):
    idx = inputs.reshape(-1).astype(jnp.int32)
    tgt = targets.reshape(-1).astype(jnp.int32)
    fused = _make_table(emb_table, W, b)
    logits_pad, parts = _sc_gather(fused, idx, tgt)
    loss = _reduce_loss(parts)[0, 0]
    return (logits_pad[:, :V], loss)
